# Initial kernel scaffold; baseline (speedup 1.0000x reference)
import jax
import jax.numpy as jnp
from jax.experimental import pallas as pl


def kernel(x, Wr, br, W, b):
    raise NotImplementedError("write your pallas kernel here")


# speedup vs baseline: 48.2842x; 48.2842x over previous
import functools

import jax
import jax.numpy as jnp
from jax.experimental import pallas as pl

N, D_IN, D_OUT, TOPK = 8192, 1024, 1024, 8
BLOCK_ROWS = 256


def _body(x_ref, wrt_ref, br_ref, wt_ref, b_ref, o_ref):
    x = x_ref[...]
    logits = jax.lax.dot_general(
        x, wrt_ref[...], (((1,), (0,)), ((), ())),
        preferred_element_type=jnp.float32,
    ) + br_ref[...]
    t = logits
    for _ in range(TOPK - 1):
        m = jnp.max(t, axis=1, keepdims=True)
        t = jnp.where(t >= m, -jnp.inf, t)
    thresh = jnp.max(t, axis=1, keepdims=True)
    mask = (logits >= thresh).astype(jnp.float32)
    orig = jax.lax.dot_general(
        x, wt_ref[...], (((1,), (0,)), ((), ())),
        preferred_element_type=jnp.float32,
    ) + b_ref[...]
    o_ref[...] = orig * mask


@jax.jit
def kernel(x, Wr, br, W, b):
    wrt = Wr.T
    wt = W.T
    br2 = br.reshape(1, D_OUT)
    b2 = b.reshape(1, D_OUT)
    grid = (N // BLOCK_ROWS,)
    return pl.pallas_call(
        _body,
        grid=grid,
        in_specs=[
            pl.BlockSpec((BLOCK_ROWS, D_IN), lambda i: (i, 0)),
            pl.BlockSpec((D_IN, D_OUT), lambda i: (0, 0)),
            pl.BlockSpec((1, D_OUT), lambda i: (0, 0)),
            pl.BlockSpec((D_IN, D_OUT), lambda i: (0, 0)),
            pl.BlockSpec((1, D_OUT), lambda i: (0, 0)),
        ],
        out_specs=pl.BlockSpec((BLOCK_ROWS, D_OUT), lambda i: (i, 0)),
        out_shape=jax.ShapeDtypeStruct((N, D_OUT), jnp.float32),
    )(x, wrt, br2, wt, b2)
